# baseline (speedup 1.0000x reference)
import functools

import jax
import jax.numpy as jnp
from jax import lax
from jax.experimental import pallas as pl
from jax.experimental.pallas import tpu as pltpu
from jax.experimental.pallas import tpu_sc as plsc

N = 10000
E = 160000
D = 256
H = 128
NC = 2
NS = 16
CHUNK = 128
CPT = 80
EPT = CPT * CHUNK
E_PAD = NS * EPT
N_PAD = 10240
RPT = N_PAD // NS
ZR = 64
RB = 10
BN = N // RB

_mesh = plsc.VectorSubcoreMesh(
    core_axis_name="c", subcore_axis_name="s", num_cores=NC, num_subcores=NS)


def _deg_body(row_hbm, col_hbm, deginv_hbm, beta_hbm,
              rbuf, cbuf, cnt_ns, cnt_sf, red_ns, red_sf, dib, beb,
              sh_ns, sh_sf):
    c = lax.axis_index("c")
    s = lax.axis_index("s")

    @pl.when(c == 0)
    def _work():
        zeros16 = jnp.zeros((16,), jnp.float32)
        ones16 = jnp.ones((16,), jnp.float32)

        def zero_cnt(j, carry):
            cnt_ns[pl.ds(j * 16, 16)] = zeros16
            cnt_sf[pl.ds(j * 16, 16)] = zeros16
            return carry
        lax.fori_loop(0, N_PAD // 16, zero_cnt, 0)
        pltpu.sync_copy(row_hbm.at[s], rbuf)
        pltpu.sync_copy(col_hbm.at[s], cbuf)

        def scan_chunk(ci, carry):
            def scan_vec(cj, carry2):
                r16 = rbuf[ci, pl.ds(cj * 16, 16)]
                c16 = cbuf[ci, pl.ds(cj * 16, 16)]
                m_ns = r16 != c16
                plsc.addupdate_scatter(cnt_ns, [c16], ones16, mask=m_ns)
                plsc.addupdate_scatter(cnt_sf, [c16], ones16,
                                       mask=jnp.logical_not(m_ns))
                return carry2
            return lax.fori_loop(0, CHUNK // 16, scan_vec, carry)
        lax.fori_loop(0, CPT, scan_chunk, 0)

        pltpu.sync_copy(cnt_ns, sh_ns.at[s])
        pltpu.sync_copy(cnt_sf, sh_sf.at[s])
        plsc.subcore_barrier()
        base = 640 * s
        pltpu.sync_copy(sh_ns.at[:, pl.ds(base, 640)], red_ns)
        pltpu.sync_copy(sh_sf.at[:, pl.ds(base, 640)], red_sf)
        for i in range(40):
            ns = red_ns[0, pl.ds(16 * i, 16)]
            sf = red_sf[0, pl.ds(16 * i, 16)]
            for k in range(1, NS):
                ns = ns + red_ns[k, pl.ds(16 * i, 16)]
                sf = sf + red_sf[k, pl.ds(16 * i, 16)]
            di = 1.0 / (1.0 + ns)
            dib[pl.ds(16 * i, 16)] = di
            beb[pl.ds(16 * i, 16)] = di * (1.0 - sf)
        pltpu.sync_copy(dib, deginv_hbm.at[pl.ds(base, 640)])
        pltpu.sync_copy(beb, beta_hbm.at[pl.ds(base, 640)])


_deg_kernel = pl.kernel(
    _deg_body,
    out_type=(jax.ShapeDtypeStruct((N_PAD,), jnp.float32),
              jax.ShapeDtypeStruct((N_PAD,), jnp.float32)),
    mesh=_mesh,
    scratch_types=(
        pltpu.VMEM((CPT, CHUNK), jnp.int32),
        pltpu.VMEM((CPT, CHUNK), jnp.int32),
        pltpu.VMEM((N_PAD,), jnp.float32),
        pltpu.VMEM((N_PAD,), jnp.float32),
        pltpu.VMEM((NS, 640), jnp.float32),
        pltpu.VMEM((NS, 640), jnp.float32),
        pltpu.VMEM((640,), jnp.float32),
        pltpu.VMEM((640,), jnp.float32),
        pltpu.VMEM_SHARED((NS, N_PAD), jnp.float32),
        pltpu.VMEM_SHARED((NS, N_PAD), jnp.float32),
    ),
    compiler_params=pltpu.CompilerParams(needs_layout_passes=False),
)


def _agg_body(y2_hbm, row2_hbm, col_hbm, zeros_hbm, s_hbm,
              rbuf, cbuf, gbuf, zdbuf, acc, sem, dsem):
    c = lax.axis_index("c")
    s = lax.axis_index("s")
    base = jnp.minimum(s * RPT, N - RPT)
    pltpu.sync_copy(zeros_hbm, zdbuf)
    for k in range(RPT // ZR):
        pltpu.sync_copy(zdbuf, acc.at[pl.ds(base + ZR * k, ZR)])
    pltpu.sync_copy(row2_hbm.at[c, s], rbuf)
    pltpu.sync_copy(col_hbm.at[s], cbuf)
    plsc.subcore_barrier()

    def body(j, carry):
        pltpu.async_copy(y2_hbm.at[rbuf.at[j]], gbuf, sem).wait()
        pltpu.sync_copy(gbuf, acc.at[cbuf.at[j]], add=True)
        return carry
    lax.fori_loop(0, CPT, body, 0)
    plsc.subcore_barrier()

    for k in range(RPT // ZR):
        pltpu.sync_copy(acc.at[pl.ds(base + ZR * k, ZR)], zdbuf)
        pltpu.sync_copy(zdbuf, s_hbm.at[c, pl.ds(base + ZR * k, ZR)])


_agg_kernel = pl.kernel(
    _agg_body,
    out_type=jax.ShapeDtypeStruct((NC, N, H), jnp.float32),
    mesh=_mesh,
    scratch_types=(
        pltpu.VMEM((CPT, CHUNK), jnp.int32),
        pltpu.VMEM((CPT, CHUNK), jnp.int32),
        pltpu.VMEM((CHUNK, H), jnp.float32),
        pltpu.VMEM((ZR, H), jnp.float32),
        pltpu.VMEM_SHARED((N_PAD, H), jnp.float32),
        pltpu.SemaphoreType.DMA,
        pltpu.SemaphoreType.DMA,
    ),
    compiler_params=pltpu.CompilerParams(needs_layout_passes=False),
)


def _mm0_body(x_ref, w_ref, b_ref, y2_out, z_out):
    a = x_ref[...]
    yz = jnp.dot(a, w_ref[...], preferred_element_type=jnp.float32) + b_ref[...]
    y2_out[0] = yz[:, :H]
    y2_out[1] = yz[:, H:D]
    z_out[...] = yz[:, D:]


def _mm_body(s_ref, y_ref, z_ref, di_ref, be_ref, w_ref, b_ref,
             y2_out, z_out):
    sc = jnp.concatenate([s_ref[0], s_ref[1]], axis=1)
    yc = jnp.concatenate([y_ref[0], y_ref[1]], axis=1)
    a = jnp.maximum(di_ref[...] * sc + be_ref[...] * yc + z_ref[...], 0.0)
    yz = jnp.dot(a, w_ref[...], preferred_element_type=jnp.float32) + b_ref[...]
    y2_out[0] = yz[:, :H]
    y2_out[1] = yz[:, H:D]
    z_out[...] = yz[:, D:]


def _fin_body(s_ref, y_ref, z_ref, di_ref, be_ref, o_ref):
    sc = jnp.concatenate([s_ref[0], s_ref[1]], axis=1)
    yc = jnp.concatenate([y_ref[0], y_ref[1]], axis=1)
    h = jnp.maximum(di_ref[...] * sc + be_ref[...] * yc + z_ref[...], 0.0)
    m = jnp.max(h, axis=1, keepdims=True)
    lse = jnp.log(jnp.sum(jnp.exp(h - m), axis=1, keepdims=True)) + m
    o_ref[...] = h - lse


_spec_s = pl.BlockSpec((NC, BN, H), lambda i: (0, i, 0))
_spec_x = pl.BlockSpec((BN, D), lambda i: (i, 0))
_spec_v = pl.BlockSpec((BN, 1), lambda i: (i, 0))
_spec_w = pl.BlockSpec((D, 2 * D), lambda i: (0, 0))
_spec_b = pl.BlockSpec((1, 2 * D), lambda i: (0, 0))
_out_yz = [jax.ShapeDtypeStruct((NC, N, H), jnp.float32),
           jax.ShapeDtypeStruct((N, D), jnp.float32)]


def _mm0_call(x, wcat, bcat):
    return pl.pallas_call(
        _mm0_body,
        grid=(RB,),
        in_specs=[_spec_x, _spec_w, _spec_b],
        out_specs=[_spec_s, _spec_x],
        out_shape=_out_yz,
    )(x, wcat, bcat)


def _mm_call(S, y2, z, di, be, wcat, bcat):
    return pl.pallas_call(
        _mm_body,
        grid=(RB,),
        in_specs=[_spec_s, _spec_s, _spec_x, _spec_v, _spec_v,
                  _spec_w, _spec_b],
        out_specs=[_spec_s, _spec_x],
        out_shape=_out_yz,
    )(S, y2, z, di, be, wcat, bcat)


def _fin_call(S, y2, z, di, be):
    return pl.pallas_call(
        _fin_body,
        grid=(RB,),
        in_specs=[_spec_s, _spec_s, _spec_x, _spec_v, _spec_v],
        out_specs=_spec_x,
        out_shape=jax.ShapeDtypeStruct((N, D), jnp.float32),
    )(S, y2, z, di, be)


def kernel(x, edge_index, edge_attr, W_out0, b_out0, W_root0,
           W_out1, b_out1, W_root1, W_out2, b_out2, W_root2):
    row = edge_index[0]
    col = edge_index[1]
    pad = E_PAD - E
    row_p = jnp.concatenate([row, jnp.zeros((pad,), jnp.int32)])
    col_p = jnp.concatenate([col, jnp.full((pad,), N_PAD - 1, jnp.int32)])
    row_g = row_p.reshape(NS, CPT, CHUNK)
    col_g = col_p.reshape(NS, CPT, CHUNK)
    row2_g = jnp.stack([row_g, row_g + N])
    zeros_blk = jnp.zeros((ZR, H), jnp.float32)

    deg1d, beta1d = _deg_kernel(row_g, col_g)
    di = deg1d[:N].reshape(N, 1)
    be = beta1d[:N].reshape(N, 1)

    zeros_d = jnp.zeros((D,), jnp.float32)
    layers = [
        (jnp.concatenate([W_out0, W_root0], axis=1),
         jnp.concatenate([zeros_d, b_out0]).reshape(1, 2 * D)),
        (jnp.concatenate([W_out1, W_root1], axis=1),
         jnp.concatenate([zeros_d, b_out1]).reshape(1, 2 * D)),
        (jnp.concatenate([W_out2, W_root2], axis=1),
         jnp.concatenate([zeros_d, b_out2]).reshape(1, 2 * D)),
    ]

    y2, z = _mm0_call(x, *layers[0])
    for li in (1, 2):
        S = _agg_kernel(y2.reshape(NC * N, H), row2_g, col_g, zeros_blk)
        y2, z = _mm_call(S, y2, z, di, be, *layers[li])
    S = _agg_kernel(y2.reshape(NC * N, H), row2_g, col_g, zeros_blk)
    return _fin_call(S, y2, z, di, be)


# speedup vs baseline: 1.3622x; 1.3622x over previous
import functools

import jax
import jax.numpy as jnp
from jax import lax
from jax.experimental import pallas as pl
from jax.experimental.pallas import tpu as pltpu
from jax.experimental.pallas import tpu_sc as plsc

N = 10000
E = 160000
D = 256
H = 128
NC = 2
NS = 16
CHUNK = 128
CPT = 79
EPT = CPT * CHUNK
E_PAD = NS * EPT
N_PAD = 10240
RPT = N_PAD // NS
ZR = 64
RB = 10
BN = N // RB

_mesh = plsc.VectorSubcoreMesh(
    core_axis_name="c", subcore_axis_name="s", num_cores=NC, num_subcores=NS)


def _deg_body(row_hbm, col_hbm, deginv_hbm, beta_hbm,
              rbuf, cbuf, cnt_ns, cnt_sf, red_ns, red_sf, dib, beb,
              sh_ns, sh_sf):
    c = lax.axis_index("c")
    s = lax.axis_index("s")

    @pl.when(c == 0)
    def _work():
        zeros16 = jnp.zeros((16,), jnp.float32)
        ones16 = jnp.ones((16,), jnp.float32)

        def zero_cnt(j, carry):
            cnt_ns[pl.ds(j * 16, 16)] = zeros16
            cnt_sf[pl.ds(j * 16, 16)] = zeros16
            return carry
        lax.fori_loop(0, N_PAD // 16, zero_cnt, 0)
        pltpu.sync_copy(row_hbm.at[s], rbuf)
        pltpu.sync_copy(col_hbm.at[s], cbuf)

        def scan_chunk(ci, carry):
            def scan_vec(cj, carry2):
                r16 = rbuf[ci, pl.ds(cj * 16, 16)]
                c16 = cbuf[ci, pl.ds(cj * 16, 16)]
                m_ns = r16 != c16
                plsc.addupdate_scatter(cnt_ns, [c16], ones16, mask=m_ns)
                plsc.addupdate_scatter(cnt_sf, [c16], ones16,
                                       mask=jnp.logical_not(m_ns))
                return carry2
            return lax.fori_loop(0, CHUNK // 16, scan_vec, carry)
        lax.fori_loop(0, CPT, scan_chunk, 0)

        pltpu.sync_copy(cnt_ns, sh_ns.at[s])
        pltpu.sync_copy(cnt_sf, sh_sf.at[s])
        plsc.subcore_barrier()
        base = 640 * s
        pltpu.sync_copy(sh_ns.at[:, pl.ds(base, 640)], red_ns)
        pltpu.sync_copy(sh_sf.at[:, pl.ds(base, 640)], red_sf)
        for i in range(40):
            ns = red_ns[0, pl.ds(16 * i, 16)]
            sf = red_sf[0, pl.ds(16 * i, 16)]
            for k in range(1, NS):
                ns = ns + red_ns[k, pl.ds(16 * i, 16)]
                sf = sf + red_sf[k, pl.ds(16 * i, 16)]
            di = 1.0 / (1.0 + ns)
            dib[pl.ds(16 * i, 16)] = di
            beb[pl.ds(16 * i, 16)] = di * (1.0 - sf)
        pltpu.sync_copy(dib, deginv_hbm.at[pl.ds(base, 640)])
        pltpu.sync_copy(beb, beta_hbm.at[pl.ds(base, 640)])


_deg_kernel = pl.kernel(
    _deg_body,
    out_type=(jax.ShapeDtypeStruct((N_PAD,), jnp.float32),
              jax.ShapeDtypeStruct((N_PAD,), jnp.float32)),
    mesh=_mesh,
    scratch_types=(
        pltpu.VMEM((CPT, CHUNK), jnp.int32),
        pltpu.VMEM((CPT, CHUNK), jnp.int32),
        pltpu.VMEM((N_PAD,), jnp.float32),
        pltpu.VMEM((N_PAD,), jnp.float32),
        pltpu.VMEM((NS, 640), jnp.float32),
        pltpu.VMEM((NS, 640), jnp.float32),
        pltpu.VMEM((640,), jnp.float32),
        pltpu.VMEM((640,), jnp.float32),
        pltpu.VMEM_SHARED((NS, N_PAD), jnp.float32),
        pltpu.VMEM_SHARED((NS, N_PAD), jnp.float32),
    ),
    compiler_params=pltpu.CompilerParams(needs_layout_passes=False),
)


def _agg_body(y2_hbm, row2_hbm, col_hbm, zeros_hbm, s_hbm,
              rbuf, cbuf, gbuf, zdbuf, acc, sem):
    c = lax.axis_index("c")
    s = lax.axis_index("s")
    base = jnp.minimum(s * RPT, N - RPT)
    pltpu.sync_copy(zeros_hbm, zdbuf)
    for k in range(RPT // ZR):
        pltpu.sync_copy(zdbuf, acc.at[pl.ds(base + ZR * k, ZR)])
    pltpu.sync_copy(row2_hbm.at[c, s], rbuf)
    pltpu.sync_copy(col_hbm.at[s], cbuf)
    plsc.subcore_barrier()

    def body(j, carry):
        pltpu.async_copy(y2_hbm.at[rbuf.at[j]], gbuf, sem).wait()
        pltpu.sync_copy(gbuf, acc.at[cbuf.at[j]], add=True)
        return carry
    lax.fori_loop(0, CPT, body, 0)
    plsc.subcore_barrier()

    for k in range(RPT // ZR):
        pltpu.sync_copy(acc.at[pl.ds(base + ZR * k, ZR)], zdbuf)
        pltpu.sync_copy(zdbuf, s_hbm.at[c, pl.ds(base + ZR * k, ZR)])


_agg_kernel = pl.kernel(
    _agg_body,
    out_type=jax.ShapeDtypeStruct((NC, N, H), jnp.float32),
    mesh=_mesh,
    scratch_types=(
        pltpu.VMEM((CPT, CHUNK), jnp.int32),
        pltpu.VMEM((CPT, CHUNK), jnp.int32),
        pltpu.VMEM((CHUNK, H), jnp.float32),
        pltpu.VMEM((ZR, H), jnp.float32),
        pltpu.VMEM_SHARED((N_PAD, H), jnp.float32),
        pltpu.SemaphoreType.DMA,
    ),
    compiler_params=pltpu.CompilerParams(needs_layout_passes=False),
)


def _mm_body(first, x_ref, s_ref, y_ref, z_ref, di_ref, be_ref, w_ref, b_ref,
             y2_out, z_out):
    if first:
        a = x_ref[...]
    else:
        sc = jnp.concatenate([s_ref[0], s_ref[1]], axis=1).astype(jnp.float32)
        yc = jnp.concatenate([y_ref[0], y_ref[1]], axis=1).astype(jnp.float32)
        a = jnp.maximum(di_ref[...] * sc + be_ref[...] * yc + z_ref[...], 0.0)
    yz = jnp.dot(a, w_ref[...], preferred_element_type=jnp.float32) + b_ref[...]
    y2_out[0] = yz[:, :H].astype(jnp.float32)
    y2_out[1] = yz[:, H:D].astype(jnp.float32)
    z_out[...] = yz[:, D:]


def _fin_body(s_ref, y_ref, z_ref, di_ref, be_ref, o_ref):
    sc = jnp.concatenate([s_ref[0], s_ref[1]], axis=1).astype(jnp.float32)
    yc = jnp.concatenate([y_ref[0], y_ref[1]], axis=1).astype(jnp.float32)
    h = jnp.maximum(di_ref[...] * sc + be_ref[...] * yc + z_ref[...], 0.0)
    m = jnp.max(h, axis=1, keepdims=True)
    lse = jnp.log(jnp.sum(jnp.exp(h - m), axis=1, keepdims=True)) + m
    o_ref[...] = h - lse


_spec_s = pl.BlockSpec((NC, BN, H), lambda i: (0, i, 0))
_spec_x = pl.BlockSpec((BN, D), lambda i: (i, 0))
_spec_v = pl.BlockSpec((BN, 1), lambda i: (i, 0))
_spec_w = pl.BlockSpec((D, 2 * D), lambda i: (0, 0))
_spec_b = pl.BlockSpec((1, 2 * D), lambda i: (0, 0))


def _mm_call(first, x, S, y2, z, di, be, wcat, bcat):
    return pl.pallas_call(
        functools.partial(_mm_body, first),
        grid=(RB,),
        in_specs=[_spec_x, _spec_s, _spec_s, _spec_x, _spec_v, _spec_v,
                  _spec_w, _spec_b],
        out_specs=[_spec_s, _spec_x],
        out_shape=[jax.ShapeDtypeStruct((NC, N, H), jnp.float32),
                   jax.ShapeDtypeStruct((N, D), jnp.float32)],
    )(x, S, y2, z, di, be, wcat, bcat)


def _fin_call(S, y2, z, di, be):
    return pl.pallas_call(
        _fin_body,
        grid=(RB,),
        in_specs=[_spec_s, _spec_s, _spec_x, _spec_v, _spec_v],
        out_specs=_spec_x,
        out_shape=jax.ShapeDtypeStruct((N, D), jnp.float32),
    )(S, y2, z, di, be)


def kernel(x, edge_index, edge_attr, W_out0, b_out0, W_root0,
           W_out1, b_out1, W_root1, W_out2, b_out2, W_root2):
    row = edge_index[0]
    col = edge_index[1]
    pad = E_PAD - E
    row_p = jnp.concatenate([row, jnp.zeros((pad,), jnp.int32)])
    col_p = jnp.concatenate([col, jnp.full((pad,), N_PAD - 1, jnp.int32)])
    row_g = row_p.reshape(NS, CPT, CHUNK)
    col_g = col_p.reshape(NS, CPT, CHUNK)
    row2_g = jnp.stack([row_g, row_g + N])
    zeros_blk = jnp.zeros((ZR, H), jnp.float32)

    deg1d, beta1d = _deg_kernel(row_g, col_g)
    di = deg1d[:N].reshape(N, 1)
    be = beta1d[:N].reshape(N, 1)

    zeros_d = jnp.zeros((D,), jnp.float32)
    layers = [
        (jnp.concatenate([W_out0, W_root0], axis=1),
         jnp.concatenate([zeros_d, b_out0]).reshape(1, 2 * D)),
        (jnp.concatenate([W_out1, W_root1], axis=1),
         jnp.concatenate([zeros_d, b_out1]).reshape(1, 2 * D)),
        (jnp.concatenate([W_out2, W_root2], axis=1),
         jnp.concatenate([zeros_d, b_out2]).reshape(1, 2 * D)),
    ]

    dummy_s = jnp.zeros((NC, N, H), jnp.float32)
    y2, z = _mm_call(True, x, dummy_s, dummy_s, x, di, be, *layers[0])
    for li in (1, 2):
        S = _agg_kernel(y2.reshape(NC * N, H), row2_g, col_g, zeros_blk)
        y2, z = _mm_call(False, x, S, y2, z, di, be, *layers[li])
    S = _agg_kernel(y2.reshape(NC * N, H), row2_g, col_g, zeros_blk)
    return _fin_call(S, y2, z, di, be)


# speedup vs baseline: 1.3869x; 1.0181x over previous
import functools

import jax
import jax.numpy as jnp
from jax import lax
from jax.experimental import pallas as pl
from jax.experimental.pallas import tpu as pltpu
from jax.experimental.pallas import tpu_sc as plsc

N = 10000
E = 160000
D = 256
H = 128
NC = 2
NS = 16
CHUNK = 128
CPT = 79
EPT = CPT * CHUNK
E_PAD = NS * EPT
N_PAD = 10240
RPT = N_PAD // NS
ZR = 64
RB = 10
BN = N // RB

_mesh = plsc.VectorSubcoreMesh(
    core_axis_name="c", subcore_axis_name="s", num_cores=NC, num_subcores=NS)


def _deg_body(row_hbm, col_hbm, deginv_hbm, beta_hbm,
              rbuf, cbuf, cnt_ns, cnt_sf, red_ns, red_sf, dib, beb,
              sh_ns, sh_sf):
    c = lax.axis_index("c")
    s = lax.axis_index("s")

    @pl.when(c == 0)
    def _work():
        zeros16 = jnp.zeros((16,), jnp.float32)
        ones16 = jnp.ones((16,), jnp.float32)

        def zero_cnt(j, carry):
            cnt_ns[pl.ds(j * 16, 16)] = zeros16
            cnt_sf[pl.ds(j * 16, 16)] = zeros16
            return carry
        lax.fori_loop(0, N_PAD // 16, zero_cnt, 0)
        pltpu.sync_copy(row_hbm.at[s], rbuf)
        pltpu.sync_copy(col_hbm.at[s], cbuf)

        def scan_chunk(ci, carry):
            def scan_vec(cj, carry2):
                r16 = rbuf[ci, pl.ds(cj * 16, 16)]
                c16 = cbuf[ci, pl.ds(cj * 16, 16)]
                m_ns = r16 != c16
                plsc.addupdate_scatter(cnt_ns, [c16], ones16, mask=m_ns)
                plsc.addupdate_scatter(cnt_sf, [c16], ones16,
                                       mask=jnp.logical_not(m_ns))
                return carry2
            return lax.fori_loop(0, CHUNK // 16, scan_vec, carry)
        lax.fori_loop(0, CPT, scan_chunk, 0)

        pltpu.sync_copy(cnt_ns, sh_ns.at[s])
        pltpu.sync_copy(cnt_sf, sh_sf.at[s])
        plsc.subcore_barrier()
        base = 640 * s
        pltpu.sync_copy(sh_ns.at[:, pl.ds(base, 640)], red_ns)
        pltpu.sync_copy(sh_sf.at[:, pl.ds(base, 640)], red_sf)
        for i in range(40):
            ns = red_ns[0, pl.ds(16 * i, 16)]
            sf = red_sf[0, pl.ds(16 * i, 16)]
            for k in range(1, NS):
                ns = ns + red_ns[k, pl.ds(16 * i, 16)]
                sf = sf + red_sf[k, pl.ds(16 * i, 16)]
            di = 1.0 / (1.0 + ns)
            dib[pl.ds(16 * i, 16)] = di
            beb[pl.ds(16 * i, 16)] = di * (1.0 - sf)
        pltpu.sync_copy(dib, deginv_hbm.at[pl.ds(base, 640)])
        pltpu.sync_copy(beb, beta_hbm.at[pl.ds(base, 640)])


_deg_kernel = pl.kernel(
    _deg_body,
    out_type=(jax.ShapeDtypeStruct((N_PAD,), jnp.float32),
              jax.ShapeDtypeStruct((N_PAD,), jnp.float32)),
    mesh=_mesh,
    scratch_types=(
        pltpu.VMEM((CPT, CHUNK), jnp.int32),
        pltpu.VMEM((CPT, CHUNK), jnp.int32),
        pltpu.VMEM((N_PAD,), jnp.float32),
        pltpu.VMEM((N_PAD,), jnp.float32),
        pltpu.VMEM((NS, 640), jnp.float32),
        pltpu.VMEM((NS, 640), jnp.float32),
        pltpu.VMEM((640,), jnp.float32),
        pltpu.VMEM((640,), jnp.float32),
        pltpu.VMEM_SHARED((NS, N_PAD), jnp.float32),
        pltpu.VMEM_SHARED((NS, N_PAD), jnp.float32),
    ),
    compiler_params=pltpu.CompilerParams(needs_layout_passes=False),
)


def _agg_body(y2_hbm, row2_hbm, col_hbm, zeros_hbm, s_hbm,
              rbuf, cbuf, gbuf, zdbuf, acc, sem):
    c = lax.axis_index("c")
    s = lax.axis_index("s")
    base = jnp.minimum(s * RPT, N - RPT)
    pltpu.sync_copy(zeros_hbm, zdbuf)
    for k in range(RPT // ZR):
        pltpu.sync_copy(zdbuf, acc.at[pl.ds(base + ZR * k, ZR)])
    pltpu.sync_copy(row2_hbm.at[c, s], rbuf)
    pltpu.sync_copy(col_hbm.at[s], cbuf)
    plsc.subcore_barrier()

    def body(j, carry):
        pltpu.async_copy(y2_hbm.at[rbuf.at[j]], gbuf, sem).wait()
        pltpu.sync_copy(gbuf, acc.at[cbuf.at[j]], add=True)
        return carry
    lax.fori_loop(0, CPT, body, 0)
    plsc.subcore_barrier()

    for k in range(RPT // ZR):
        pltpu.sync_copy(acc.at[pl.ds(base + ZR * k, ZR)], zdbuf)
        pltpu.sync_copy(zdbuf, s_hbm.at[c, pl.ds(base + ZR * k, ZR)])


_agg_kernel = pl.kernel(
    _agg_body,
    out_type=jax.ShapeDtypeStruct((NC, N, H), jnp.float32),
    mesh=_mesh,
    scratch_types=(
        pltpu.VMEM((CPT, CHUNK), jnp.int32),
        pltpu.VMEM((CPT, CHUNK), jnp.int32),
        pltpu.VMEM((CHUNK, H), jnp.float32),
        pltpu.VMEM((ZR, H), jnp.float32),
        pltpu.VMEM_SHARED((N_PAD, H), jnp.float32),
        pltpu.SemaphoreType.DMA,
    ),
    compiler_params=pltpu.CompilerParams(needs_layout_passes=False),
)


def _mm_body(first, x_ref, s_ref, y_ref, z_ref, di_ref, be_ref, w_ref, b_ref,
             y2_out, z_out):
    if first:
        a = x_ref[...]
    else:
        sc = jnp.concatenate([s_ref[0], s_ref[1]], axis=1).astype(jnp.float32)
        yc = jnp.concatenate([y_ref[0], y_ref[1]], axis=1).astype(jnp.float32)
        a = jnp.maximum(di_ref[...] * sc + be_ref[...] * yc + z_ref[...], 0.0)
    yz = jnp.dot(a, w_ref[...], preferred_element_type=jnp.float32) + b_ref[...]
    y2_out[0] = yz[:, :H].astype(jnp.float32)
    y2_out[1] = yz[:, H:D].astype(jnp.float32)
    z_out[...] = yz[:, D:]


def _fin_body(s_ref, y_ref, z_ref, di_ref, be_ref, o_ref):
    sc = jnp.concatenate([s_ref[0], s_ref[1]], axis=1).astype(jnp.float32)
    yc = jnp.concatenate([y_ref[0], y_ref[1]], axis=1).astype(jnp.float32)
    h = jnp.maximum(di_ref[...] * sc + be_ref[...] * yc + z_ref[...], 0.0)
    m = jnp.max(h, axis=1, keepdims=True)
    lse = jnp.log(jnp.sum(jnp.exp(h - m), axis=1, keepdims=True)) + m
    o_ref[...] = h - lse


_spec_s = pl.BlockSpec((NC, BN, H), lambda i: (0, i, 0))
_spec_x = pl.BlockSpec((BN, D), lambda i: (i, 0))
_spec_v = pl.BlockSpec((BN, 1), lambda i: (i, 0))
_spec_w = pl.BlockSpec((D, 2 * D), lambda i: (0, 0))
_spec_b = pl.BlockSpec((1, 2 * D), lambda i: (0, 0))


def _mm_call(first, x, S, y2, z, di, be, wcat, bcat):
    return pl.pallas_call(
        functools.partial(_mm_body, first),
        grid=(RB,),
        in_specs=[_spec_x, _spec_s, _spec_s, _spec_x, _spec_v, _spec_v,
                  _spec_w, _spec_b],
        out_specs=[_spec_s, _spec_x],
        out_shape=[jax.ShapeDtypeStruct((NC, N, H), jnp.float32),
                   jax.ShapeDtypeStruct((N, D), jnp.float32)],
    )(x, S, y2, z, di, be, wcat, bcat)


def _mm0_body(x_ref, w_ref, b_ref, y2_out, z_out):
    a = x_ref[...]
    yz = jnp.dot(a, w_ref[...], preferred_element_type=jnp.float32) + b_ref[...]
    y2_out[0] = yz[:, :H]
    y2_out[1] = yz[:, H:D]
    z_out[...] = yz[:, D:]


def _mm0_call(x, wcat, bcat):
    return pl.pallas_call(
        _mm0_body,
        grid=(RB,),
        in_specs=[_spec_x, _spec_w, _spec_b],
        out_specs=[_spec_s, _spec_x],
        out_shape=[jax.ShapeDtypeStruct((NC, N, H), jnp.float32),
                   jax.ShapeDtypeStruct((N, D), jnp.float32)],
    )(x, wcat, bcat)


def _fin_call(S, y2, z, di, be):
    return pl.pallas_call(
        _fin_body,
        grid=(RB,),
        in_specs=[_spec_s, _spec_s, _spec_x, _spec_v, _spec_v],
        out_specs=_spec_x,
        out_shape=jax.ShapeDtypeStruct((N, D), jnp.float32),
    )(S, y2, z, di, be)


def kernel(x, edge_index, edge_attr, W_out0, b_out0, W_root0,
           W_out1, b_out1, W_root1, W_out2, b_out2, W_root2):
    row = edge_index[0]
    col = edge_index[1]
    pad = E_PAD - E
    row_p = jnp.concatenate([row, jnp.zeros((pad,), jnp.int32)])
    col_p = jnp.concatenate([col, jnp.full((pad,), N_PAD - 1, jnp.int32)])
    row_g = row_p.reshape(NS, CPT, CHUNK)
    col_g = col_p.reshape(NS, CPT, CHUNK)
    row2_g = jnp.stack([row_g, row_g + N])
    zeros_blk = jnp.zeros((ZR, H), jnp.float32)

    deg1d, beta1d = _deg_kernel(row_g, col_g)
    di = deg1d[:N].reshape(N, 1)
    be = beta1d[:N].reshape(N, 1)

    zeros_d = jnp.zeros((D,), jnp.float32)
    layers = [
        (jnp.concatenate([W_out0, W_root0], axis=1),
         jnp.concatenate([zeros_d, b_out0]).reshape(1, 2 * D)),
        (jnp.concatenate([W_out1, W_root1], axis=1),
         jnp.concatenate([zeros_d, b_out1]).reshape(1, 2 * D)),
        (jnp.concatenate([W_out2, W_root2], axis=1),
         jnp.concatenate([zeros_d, b_out2]).reshape(1, 2 * D)),
    ]

    y2, z = _mm0_call(x, *layers[0])
    for li in (1, 2):
        S = _agg_kernel(y2.reshape(NC * N, H), row2_g, col_g, zeros_blk)
        y2, z = _mm_call(False, x, S, y2, z, di, be, *layers[li])
    S = _agg_kernel(y2.reshape(NC * N, H), row2_g, col_g, zeros_blk)
    return _fin_call(S, y2, z, di, be)
